# baseline (speedup 1.0000x reference)
import functools

import jax
import jax.numpy as jnp
from jax import lax
from jax.experimental import pallas as pl
from jax.experimental.pallas import tpu as pltpu
from jax.experimental.pallas import tpu_sc as plsc

B = 16384
L = 200
VOCAB = 1000000
HID = 32
NCP = 16
BMT = 8192
NBLK = 123
VCAP = NBLK * BMT
QROWS = BMT // 8
NC = 2
NS = 16
NW = NC * NS
BPW = B // NW
NB = 8
GROUPS = BPW // NB
IDXG = NB * L
BCHUNKS = [(0, 128), (128, 72)]
UNROLL = 10


def _table_kernel(embT_ref, w1_ref, b1_ref, w2p_ref, b2p_ref, o_ref, bias_ref):
    wc = jnp.dot(w1_ref[...], w2p_ref[...],
                 preferred_element_type=jnp.float32)
    bias_ref[...] = jnp.dot(b1_ref[...], w2p_ref[...],
                            preferred_element_type=jnp.float32) + b2p_ref[...]
    res = lax.dot_general(
        embT_ref[...], wc, (((0,), (0,)), ((), ())),
        preferred_element_type=jnp.float32)
    for q in range(8):
        o_ref[:, q * NCP:(q + 1) * NCP] = res[q * QROWS:(q + 1) * QROWS, :]


def _pool_kernel(text_hbm, tbl_hbm, bias_hbm, out_hbm,
                 ibuf0, ibuf1, rbuf0, rbuf1, out_v, bias_v,
                 isem0, isem1, gsem0, gsem1):
    wid = lax.axis_index("s") * NC + lax.axis_index("c")
    inv_l = jnp.float32(1.0 / L)
    zero = jnp.zeros((16,), jnp.float32)
    pltpu.sync_copy(bias_hbm, bias_v)
    bvec = bias_v[0, :]
    ibufs = (ibuf0, ibuf1)
    rbufs = (rbuf0, rbuf1)
    isems = (isem0, isem1)
    gsems = (gsem0, gsem1)

    lane = lax.iota(jnp.int32, 16)

    def remap(v):
        return (v & -8192) | ((v & 1023) << 3) | ((v >> 10) & 7)

    def start_idx(g, b):
        base = wid * BPW + g * NB
        pltpu.async_copy(text_hbm.at[pl.ds(base, NB)], ibufs[b], isems[b])

    def fire_gathers(b):
        pltpu.make_async_copy(text_hbm.at[pl.ds(0, NB)],
                              ibufs[b], isems[b]).wait()
        ib = ibufs[b]
        for i in range(NB):
            for s in range(12):
                v = ib[i, pl.ds(s * 16, 16)]
                ib[i, pl.ds(s * 16, 16)] = remap(v)
            v = ib[i, pl.ds(184, 16)]
            ib[i, pl.ds(184, 16)] = jnp.where(lane >= 8, remap(v), v)
        for i in range(NB):
            for off, n in BCHUNKS:
                pltpu.async_copy(tbl_hbm.at[ib.at[i, pl.ds(off, n)]],
                                 rbufs[b].at[pl.ds(i * L + off, n)], gsems[b])

    def drain_reduce(g, b):
        for i in range(NB):
            for off, n in BCHUNKS:
                pltpu.make_async_copy(
                    tbl_hbm.at[ibufs[b].at[i, pl.ds(off, n)]],
                    rbufs[b].at[pl.ds(i * L + off, n)], gsems[b]).wait()
        rbuf = rbufs[b]
        for i in range(NB):
            @pl.loop(0, L, init_carry=zero, unroll=UNROLL)
            def _acc(j, a):
                return a + rbuf[i * L + j]

            out_v[g * NB + i, :] = _acc * inv_l + bvec

    start_idx(0, 0)
    start_idx(1, 1)
    fire_gathers(0)

    @pl.loop(0, GROUPS, step=2)
    def _pair(g):
        fire_gathers(1)
        drain_reduce(g, 0)

        @pl.when(g + 2 < GROUPS)
        def _():
            start_idx(g + 2, 0)

        @pl.when(g + 2 < GROUPS)
        def _():
            fire_gathers(0)

        drain_reduce(g + 1, 1)

        @pl.when(g + 3 < GROUPS)
        def _():
            start_idx(g + 3, 1)

    pltpu.sync_copy(out_v, out_hbm.at[pl.ds(wid * BPW, BPW)])


def kernel(text, text_lengths, emb, W1, b1, W2, b2):
    del text_lengths
    text32 = text.astype(jnp.int32)
    embT = emb.T
    NCLS = b2.shape[0]
    W2p = jnp.pad(W2, ((0, 0), (0, NCP - NCLS)))
    b2p = jnp.pad(b2, (0, NCP - NCLS))

    tablec, biasc = pl.pallas_call(
        _table_kernel,
        grid=(NBLK,),
        in_specs=[
            pl.BlockSpec((HID, BMT), lambda i: (0, i)),
            pl.BlockSpec((HID, HID), lambda i: (0, 0)),
            pl.BlockSpec((1, HID), lambda i: (0, 0)),
            pl.BlockSpec((HID, NCP), lambda i: (0, 0)),
            pl.BlockSpec((1, NCP), lambda i: (0, 0)),
        ],
        out_specs=[
            pl.BlockSpec((QROWS, 128), lambda i: (i, 0)),
            pl.BlockSpec((1, NCP), lambda i: (0, 0)),
        ],
        out_shape=[
            jax.ShapeDtypeStruct((VCAP // 8, 128), jnp.float32),
            jax.ShapeDtypeStruct((1, NCP), jnp.float32),
        ],
    )(embT, W1, b1.reshape(1, HID), W2p, b2p.reshape(1, NCP))
    table2 = tablec.reshape(VCAP, NCP)

    mesh = plsc.VectorSubcoreMesh(core_axis_name="c", subcore_axis_name="s")
    pooled = pl.kernel(
        _pool_kernel,
        out_type=jax.ShapeDtypeStruct((B, NCP), jnp.float32),
        mesh=mesh,
        compiler_params=pltpu.CompilerParams(use_tc_tiling_on_sc=False),
        scratch_types=[
            pltpu.VMEM((NB, L), jnp.int32),
            pltpu.VMEM((NB, L), jnp.int32),
            pltpu.VMEM((IDXG, NCP), jnp.float32),
            pltpu.VMEM((IDXG, NCP), jnp.float32),
            pltpu.VMEM((BPW, NCP), jnp.float32),
            pltpu.VMEM((1, NCP), jnp.float32),
            pltpu.SemaphoreType.DMA,
            pltpu.SemaphoreType.DMA,
            pltpu.SemaphoreType.DMA,
            pltpu.SemaphoreType.DMA,
        ],
    )(text32, table2, biasc)

    return pooled[:, :NCLS]


# speedup vs baseline: 1.0019x; 1.0019x over previous
import functools

import jax
import jax.numpy as jnp
from jax import lax
from jax.experimental import pallas as pl
from jax.experimental.pallas import tpu as pltpu
from jax.experimental.pallas import tpu_sc as plsc

B = 16384
L = 200
VOCAB = 1000000
HID = 32
NCP = 16
BMT = 8192
NBLK = 123
VCAP = NBLK * BMT
QROWS = BMT // 8
NC = 2
NS = 16
NW = NC * NS
BPW = B // NW
NB = 16
GROUPS = BPW // NB
IDXG = NB * L
BCHUNKS = [(0, 128), (128, 72)]
UNROLL = 10


def _table_kernel(embT_ref, w1_ref, b1_ref, w2p_ref, b2p_ref, o_ref, bias_ref):
    wc = jnp.dot(w1_ref[...], w2p_ref[...],
                 preferred_element_type=jnp.float32)
    bias_ref[...] = jnp.dot(b1_ref[...], w2p_ref[...],
                            preferred_element_type=jnp.float32) + b2p_ref[...]
    res = lax.dot_general(
        embT_ref[...], wc, (((0,), (0,)), ((), ())),
        preferred_element_type=jnp.float32)
    for q in range(8):
        o_ref[:, q * NCP:(q + 1) * NCP] = res[q * QROWS:(q + 1) * QROWS, :]


def _pool_kernel(text_hbm, tbl_hbm, bias_hbm, out_hbm,
                 ibuf0, ibuf1, rbuf0, rbuf1, out_v, bias_v,
                 isem0, isem1, gsem0, gsem1):
    wid = lax.axis_index("s") * NC + lax.axis_index("c")
    inv_l = jnp.float32(1.0 / L)
    zero = jnp.zeros((16,), jnp.float32)
    pltpu.sync_copy(bias_hbm, bias_v)
    bvec = bias_v[0, :]
    ibufs = (ibuf0, ibuf1)
    rbufs = (rbuf0, rbuf1)
    isems = (isem0, isem1)
    gsems = (gsem0, gsem1)

    lane = lax.iota(jnp.int32, 16)

    def remap(v):
        return (v & -8192) | ((v & 1023) << 3) | ((v >> 10) & 7)

    def start_idx(g, b):
        base = wid * BPW + g * NB
        pltpu.async_copy(text_hbm.at[pl.ds(base, NB)], ibufs[b], isems[b])

    def fire_gathers(b):
        pltpu.make_async_copy(text_hbm.at[pl.ds(0, NB)],
                              ibufs[b], isems[b]).wait()
        ib = ibufs[b]
        for i in range(NB):
            for s in range(12):
                v = ib[i, pl.ds(s * 16, 16)]
                ib[i, pl.ds(s * 16, 16)] = remap(v)
            v = ib[i, pl.ds(184, 16)]
            ib[i, pl.ds(184, 16)] = jnp.where(lane >= 8, remap(v), v)
        for i in range(NB):
            for off, n in BCHUNKS:
                pltpu.async_copy(tbl_hbm.at[ib.at[i, pl.ds(off, n)]],
                                 rbufs[b].at[pl.ds(i * L + off, n)], gsems[b])

    def drain_reduce(g, b):
        for i in range(NB):
            for off, n in BCHUNKS:
                pltpu.make_async_copy(
                    tbl_hbm.at[ibufs[b].at[i, pl.ds(off, n)]],
                    rbufs[b].at[pl.ds(i * L + off, n)], gsems[b]).wait()
        rbuf = rbufs[b]
        for i in range(NB):
            @pl.loop(0, L // 4, init_carry=(zero, zero, zero, zero),
                     unroll=UNROLL)
            def _acc(j, carry):
                a0, a1, a2, a3 = carry
                base = i * L + j * 4
                return (a0 + rbuf[base], a1 + rbuf[base + 1],
                        a2 + rbuf[base + 2], a3 + rbuf[base + 3])

            a0, a1, a2, a3 = _acc
            out_v[g * NB + i, :] = ((a0 + a1) + (a2 + a3)) * inv_l + bvec

    start_idx(0, 0)
    start_idx(1, 1)
    fire_gathers(0)

    @pl.loop(0, GROUPS, step=2)
    def _pair(g):
        fire_gathers(1)
        drain_reduce(g, 0)

        @pl.when(g + 2 < GROUPS)
        def _():
            start_idx(g + 2, 0)

        @pl.when(g + 2 < GROUPS)
        def _():
            fire_gathers(0)

        drain_reduce(g + 1, 1)

        @pl.when(g + 3 < GROUPS)
        def _():
            start_idx(g + 3, 1)

    pltpu.sync_copy(out_v, out_hbm.at[pl.ds(wid * BPW, BPW)])


def kernel(text, text_lengths, emb, W1, b1, W2, b2):
    del text_lengths
    text32 = text.astype(jnp.int32)
    embT = emb.T
    NCLS = b2.shape[0]
    W2p = jnp.pad(W2, ((0, 0), (0, NCP - NCLS)))
    b2p = jnp.pad(b2, (0, NCP - NCLS))

    tablec, biasc = pl.pallas_call(
        _table_kernel,
        grid=(NBLK,),
        in_specs=[
            pl.BlockSpec((HID, BMT), lambda i: (0, i)),
            pl.BlockSpec((HID, HID), lambda i: (0, 0)),
            pl.BlockSpec((1, HID), lambda i: (0, 0)),
            pl.BlockSpec((HID, NCP), lambda i: (0, 0)),
            pl.BlockSpec((1, NCP), lambda i: (0, 0)),
        ],
        out_specs=[
            pl.BlockSpec((QROWS, 128), lambda i: (i, 0)),
            pl.BlockSpec((1, NCP), lambda i: (0, 0)),
        ],
        out_shape=[
            jax.ShapeDtypeStruct((VCAP // 8, 128), jnp.float32),
            jax.ShapeDtypeStruct((1, NCP), jnp.float32),
        ],
    )(embT, W1, b1.reshape(1, HID), W2p, b2p.reshape(1, NCP))
    table2 = tablec.reshape(VCAP, NCP)

    mesh = plsc.VectorSubcoreMesh(core_axis_name="c", subcore_axis_name="s")
    pooled = pl.kernel(
        _pool_kernel,
        out_type=jax.ShapeDtypeStruct((B, NCP), jnp.float32),
        mesh=mesh,
        compiler_params=pltpu.CompilerParams(use_tc_tiling_on_sc=False),
        scratch_types=[
            pltpu.VMEM((NB, L), jnp.int32),
            pltpu.VMEM((NB, L), jnp.int32),
            pltpu.VMEM((IDXG, NCP), jnp.float32),
            pltpu.VMEM((IDXG, NCP), jnp.float32),
            pltpu.VMEM((BPW, NCP), jnp.float32),
            pltpu.VMEM((1, NCP), jnp.float32),
            pltpu.SemaphoreType.DMA,
            pltpu.SemaphoreType.DMA,
            pltpu.SemaphoreType.DMA,
            pltpu.SemaphoreType.DMA,
        ],
    )(text32, table2, biasc)

    return pooled[:, :NCLS]


# speedup vs baseline: 1.0057x; 1.0038x over previous
import functools

import jax
import jax.numpy as jnp
from jax import lax
from jax.experimental import pallas as pl
from jax.experimental.pallas import tpu as pltpu
from jax.experimental.pallas import tpu_sc as plsc

B = 16384
L = 200
VOCAB = 1000000
HID = 32
NCP = 16
BMT = 8192
NBLK = 123
VCAP = NBLK * BMT
QROWS = BMT // 8
NC = 2
NS = 16
NW = NC * NS
BPW = B // NW
NB = 16
GROUPS = BPW // NB
IDXG = NB * L
NCHUNK = IDXG // 128
UNROLL = 10


def _table_kernel(embT_ref, w1_ref, b1_ref, w2p_ref, b2p_ref, o_ref, bias_ref):
    wc = jnp.dot(w1_ref[...], w2p_ref[...],
                 preferred_element_type=jnp.float32)
    bias_ref[...] = jnp.dot(b1_ref[...], w2p_ref[...],
                            preferred_element_type=jnp.float32) + b2p_ref[...]
    for q in range(8):
        res_q = jnp.dot(embT_ref[:, q * QROWS:(q + 1) * QROWS].T, wc,
                        preferred_element_type=jnp.float32)
        o_ref[:, q * NCP:(q + 1) * NCP] = res_q


def _pool_kernel(text_hbm, tbl_hbm, bias_hbm, out_hbm,
                 ibuf0, ibuf1, rbuf0, rbuf1, out_v, bias_v,
                 isem0, isem1, gsem0, gsem1):
    wid = lax.axis_index("s") * NC + lax.axis_index("c")
    inv_l = jnp.float32(1.0 / L)
    zero = jnp.zeros((16,), jnp.float32)
    pltpu.sync_copy(bias_hbm, bias_v)
    bvec = bias_v[0, :]
    ibufs = (ibuf0, ibuf1)
    rbufs = (rbuf0, rbuf1)
    isems = (isem0, isem1)
    gsems = (gsem0, gsem1)

    def remap(v):
        return (v & -8192) | ((v & 1023) << 3) | ((v >> 10) & 7)

    def start_idx(g, b):
        base = (wid * BPW + g * NB) * L
        pltpu.async_copy(text_hbm.at[pl.ds(base, IDXG)], ibufs[b], isems[b])

    def fire_gathers(b):
        pltpu.make_async_copy(text_hbm.at[pl.ds(0, IDXG)],
                              ibufs[b], isems[b]).wait()
        ib = ibufs[b]
        for s in range(IDXG // 16):
            ib[pl.ds(s * 16, 16)] = remap(ib[pl.ds(s * 16, 16)])
        for c in range(NCHUNK):
            pltpu.async_copy(tbl_hbm.at[ib.at[pl.ds(c * 128, 128)]],
                             rbufs[b].at[pl.ds(c * 128, 128)], gsems[b])

    def drain_reduce(g, b):
        for c in range(NCHUNK):
            pltpu.make_async_copy(
                tbl_hbm.at[ibufs[b].at[pl.ds(c * 128, 128)]],
                rbufs[b].at[pl.ds(c * 128, 128)], gsems[b]).wait()
        rbuf = rbufs[b]
        for i in range(NB):
            @pl.loop(0, L // 4, init_carry=(zero, zero, zero, zero),
                     unroll=UNROLL)
            def _acc(j, carry):
                a0, a1, a2, a3 = carry
                base = i * L + j * 4
                return (a0 + rbuf[base], a1 + rbuf[base + 1],
                        a2 + rbuf[base + 2], a3 + rbuf[base + 3])

            a0, a1, a2, a3 = _acc
            out_v[g * NB + i, :] = ((a0 + a1) + (a2 + a3)) * inv_l + bvec

    start_idx(0, 0)
    start_idx(1, 1)
    fire_gathers(0)

    @pl.loop(0, GROUPS, step=2)
    def _pair(g):
        fire_gathers(1)
        drain_reduce(g, 0)

        @pl.when(g + 2 < GROUPS)
        def _():
            start_idx(g + 2, 0)

        @pl.when(g + 2 < GROUPS)
        def _():
            fire_gathers(0)

        drain_reduce(g + 1, 1)

        @pl.when(g + 3 < GROUPS)
        def _():
            start_idx(g + 3, 1)

    pltpu.sync_copy(out_v, out_hbm.at[pl.ds(wid * BPW, BPW)])


def kernel(text, text_lengths, emb, W1, b1, W2, b2):
    del text_lengths
    text32 = text.astype(jnp.int32).reshape(B * L)
    embT = emb.T
    NCLS = b2.shape[0]
    W2p = jnp.pad(W2, ((0, 0), (0, NCP - NCLS)))
    b2p = jnp.pad(b2, (0, NCP - NCLS))

    tablec, biasc = pl.pallas_call(
        _table_kernel,
        grid=(NBLK,),
        in_specs=[
            pl.BlockSpec((HID, BMT), lambda i: (0, i)),
            pl.BlockSpec((HID, HID), lambda i: (0, 0)),
            pl.BlockSpec((1, HID), lambda i: (0, 0)),
            pl.BlockSpec((HID, NCP), lambda i: (0, 0)),
            pl.BlockSpec((1, NCP), lambda i: (0, 0)),
        ],
        out_specs=[
            pl.BlockSpec((QROWS, 128), lambda i: (i, 0)),
            pl.BlockSpec((1, NCP), lambda i: (0, 0)),
        ],
        out_shape=[
            jax.ShapeDtypeStruct((VCAP // 8, 128), jnp.float32),
            jax.ShapeDtypeStruct((1, NCP), jnp.float32),
        ],
        compiler_params=pltpu.CompilerParams(fuse_transposed_lhs_in_matmul=True),
    )(embT, W1, b1.reshape(1, HID), W2p, b2p.reshape(1, NCP))
    table2 = tablec.reshape(VCAP, NCP)

    mesh = plsc.VectorSubcoreMesh(core_axis_name="c", subcore_axis_name="s")
    pooled = pl.kernel(
        _pool_kernel,
        out_type=jax.ShapeDtypeStruct((B, NCP), jnp.float32),
        mesh=mesh,
        compiler_params=pltpu.CompilerParams(use_tc_tiling_on_sc=False),
        scratch_types=[
            pltpu.VMEM((IDXG,), jnp.int32),
            pltpu.VMEM((IDXG,), jnp.int32),
            pltpu.VMEM((IDXG, NCP), jnp.float32),
            pltpu.VMEM((IDXG, NCP), jnp.float32),
            pltpu.VMEM((BPW, NCP), jnp.float32),
            pltpu.VMEM((1, NCP), jnp.float32),
            pltpu.SemaphoreType.DMA,
            pltpu.SemaphoreType.DMA,
            pltpu.SemaphoreType.DMA,
            pltpu.SemaphoreType.DMA,
        ],
    )(text32, table2, biasc)

    return pooled[:, :NCLS]


# speedup vs baseline: 1.1501x; 1.1436x over previous
import functools

import jax
import jax.numpy as jnp
from jax import lax
from jax.experimental import pallas as pl
from jax.experimental.pallas import tpu as pltpu
from jax.experimental.pallas import tpu_sc as plsc

B = 16384
L = 200
VOCAB = 1000000
HID = 32
NCP = 16
BMT = 8192
NBLK = 123
VCAP = NBLK * BMT
QROWS = BMT // 8
NC = 2
NS = 16
NW = NC * NS
BPW = B // NW
NB = 16
GROUPS = BPW // NB
IDXG = NB * L
NCHUNK = IDXG // 128
UNROLL = 10


def _table_kernel(embT_ref, w1_ref, b1_ref, w2p_ref, b2p_ref, o_ref, bias_ref):
    wc = jnp.dot(w1_ref[...], w2p_ref[...],
                 preferred_element_type=jnp.float32)
    bias_ref[...] = jnp.dot(b1_ref[...], w2p_ref[...],
                            preferred_element_type=jnp.float32) + b2p_ref[...]
    wcb = wc.astype(jnp.bfloat16)
    for q in range(8):
        eb = embT_ref[:, q * QROWS:(q + 1) * QROWS].astype(jnp.bfloat16)
        res_q = jnp.dot(eb.T, wcb,
                        preferred_element_type=jnp.float32)
        o_ref[:, q * NCP:(q + 1) * NCP] = res_q


def _pool_kernel(text_hbm, tbl_hbm, bias_hbm, out_hbm,
                 ibuf0, ibuf1, rbuf0, rbuf1, out_v, bias_v,
                 isem0, isem1, gsem0, gsem1):
    wid = lax.axis_index("s") * NC + lax.axis_index("c")
    inv_l = jnp.float32(1.0 / L)
    zero = jnp.zeros((16,), jnp.float32)
    pltpu.sync_copy(bias_hbm, bias_v)
    bvec = bias_v[0, :]
    ibufs = (ibuf0, ibuf1)
    rbufs = (rbuf0, rbuf1)
    isems = (isem0, isem1)
    gsems = (gsem0, gsem1)

    def remap(v):
        return (v & -8192) | ((v & 1023) << 3) | ((v >> 10) & 7)

    def start_idx(g, b):
        base = (wid * BPW + g * NB) * L
        pltpu.async_copy(text_hbm.at[pl.ds(base, IDXG)], ibufs[b], isems[b])

    def fire_gathers(b):
        pltpu.make_async_copy(text_hbm.at[pl.ds(0, IDXG)],
                              ibufs[b], isems[b]).wait()
        ib = ibufs[b]
        for s in range(IDXG // 16):
            ib[pl.ds(s * 16, 16)] = remap(ib[pl.ds(s * 16, 16)])
        for c in range(NCHUNK):
            pltpu.async_copy(tbl_hbm.at[ib.at[pl.ds(c * 128, 128)]],
                             rbufs[b].at[pl.ds(c * 128, 128)], gsems[b])

    def drain_reduce(g, b):
        for c in range(NCHUNK):
            pltpu.make_async_copy(
                tbl_hbm.at[ibufs[b].at[pl.ds(c * 128, 128)]],
                rbufs[b].at[pl.ds(c * 128, 128)], gsems[b]).wait()
        rbuf = rbufs[b]
        for i in range(NB):
            @pl.loop(0, L // 4, init_carry=(zero, zero, zero, zero),
                     unroll=UNROLL)
            def _acc(j, carry):
                a0, a1, a2, a3 = carry
                base = i * L + j * 4
                return (a0 + rbuf[base], a1 + rbuf[base + 1],
                        a2 + rbuf[base + 2], a3 + rbuf[base + 3])

            a0, a1, a2, a3 = _acc
            out_v[g * NB + i, :] = ((a0 + a1) + (a2 + a3)) * inv_l + bvec

    start_idx(0, 0)
    start_idx(1, 1)
    fire_gathers(0)

    @pl.loop(0, GROUPS, step=2)
    def _pair(g):
        fire_gathers(1)
        drain_reduce(g, 0)

        @pl.when(g + 2 < GROUPS)
        def _():
            start_idx(g + 2, 0)

        @pl.when(g + 2 < GROUPS)
        def _():
            fire_gathers(0)

        drain_reduce(g + 1, 1)

        @pl.when(g + 3 < GROUPS)
        def _():
            start_idx(g + 3, 1)

    pltpu.sync_copy(out_v, out_hbm.at[pl.ds(wid * BPW, BPW)])


def kernel(text, text_lengths, emb, W1, b1, W2, b2):
    del text_lengths
    text32 = text.astype(jnp.int32).reshape(B * L)
    embT = emb.T
    NCLS = b2.shape[0]
    W2p = jnp.pad(W2, ((0, 0), (0, NCP - NCLS)))
    b2p = jnp.pad(b2, (0, NCP - NCLS))

    tablec, biasc = pl.pallas_call(
        _table_kernel,
        grid=(NBLK,),
        in_specs=[
            pl.BlockSpec((HID, BMT), lambda i: (0, i)),
            pl.BlockSpec((HID, HID), lambda i: (0, 0)),
            pl.BlockSpec((1, HID), lambda i: (0, 0)),
            pl.BlockSpec((HID, NCP), lambda i: (0, 0)),
            pl.BlockSpec((1, NCP), lambda i: (0, 0)),
        ],
        out_specs=[
            pl.BlockSpec((QROWS, 128), lambda i: (i, 0)),
            pl.BlockSpec((1, NCP), lambda i: (0, 0)),
        ],
        out_shape=[
            jax.ShapeDtypeStruct((VCAP // 8, 128), jnp.float32),
            jax.ShapeDtypeStruct((1, NCP), jnp.float32),
        ],
        compiler_params=pltpu.CompilerParams(fuse_transposed_lhs_in_matmul=True),
    )(embT, W1, b1.reshape(1, HID), W2p, b2p.reshape(1, NCP))
    table2 = tablec.reshape(VCAP, NCP)

    mesh = plsc.VectorSubcoreMesh(core_axis_name="c", subcore_axis_name="s")
    pooled = pl.kernel(
        _pool_kernel,
        out_type=jax.ShapeDtypeStruct((B, NCP), jnp.float32),
        mesh=mesh,
        compiler_params=pltpu.CompilerParams(use_tc_tiling_on_sc=False),
        scratch_types=[
            pltpu.VMEM((IDXG,), jnp.int32),
            pltpu.VMEM((IDXG,), jnp.int32),
            pltpu.VMEM((IDXG, NCP), jnp.float32),
            pltpu.VMEM((IDXG, NCP), jnp.float32),
            pltpu.VMEM((BPW, NCP), jnp.float32),
            pltpu.VMEM((1, NCP), jnp.float32),
            pltpu.SemaphoreType.DMA,
            pltpu.SemaphoreType.DMA,
            pltpu.SemaphoreType.DMA,
            pltpu.SemaphoreType.DMA,
        ],
    )(text32, table2, biasc)

    return pooled[:, :NCLS]


# speedup vs baseline: 1.1802x; 1.0262x over previous
import functools

import jax
import jax.numpy as jnp
from jax import lax
from jax.experimental import pallas as pl
from jax.experimental.pallas import tpu as pltpu
from jax.experimental.pallas import tpu_sc as plsc

B = 16384
L = 200
VOCAB = 1000000
HID = 32
NCP = 16
BMT = 16384
NBLK = 62
VCAP = NBLK * BMT
QROWS = BMT // 8
NC = 2
NS = 16
NW = NC * NS
BPW = B // NW
NB = 16
GROUPS = BPW // NB
IDXG = NB * L
NCHUNK = IDXG // 128
UNROLL = 10


def _table_kernel(embT_ref, w1_ref, b1_ref, w2p_ref, b2p_ref, o_ref, bias_ref):
    wc = jnp.dot(w1_ref[...], w2p_ref[...],
                 preferred_element_type=jnp.float32)
    bias_ref[...] = jnp.dot(b1_ref[...], w2p_ref[...],
                            preferred_element_type=jnp.float32) + b2p_ref[...]
    wcb = wc.astype(jnp.bfloat16)
    for q in range(8):
        eb = embT_ref[:, q * QROWS:(q + 1) * QROWS].astype(jnp.bfloat16)
        res_q = jnp.dot(eb.T, wcb,
                        preferred_element_type=jnp.float32)
        o_ref[:, q * NCP:(q + 1) * NCP] = res_q


def _pool_kernel(text_hbm, tbl_hbm, bias_hbm, out_hbm,
                 ibuf0, ibuf1, rbuf0, rbuf1, out_v, bias_v,
                 isem0, isem1, gsem0, gsem1):
    wid = lax.axis_index("s") * NC + lax.axis_index("c")
    inv_l = jnp.float32(1.0 / L)
    zero = jnp.zeros((16,), jnp.float32)
    pltpu.sync_copy(bias_hbm, bias_v)
    bvec = bias_v[0, :]
    ibufs = (ibuf0, ibuf1)
    rbufs = (rbuf0, rbuf1)
    isems = (isem0, isem1)
    gsems = (gsem0, gsem1)

    def remap(v):
        return (v & -BMT) | ((v & (QROWS - 1)) << 3) | ((v >> 11) & 7)

    def start_idx(g, b):
        base = (wid * BPW + g * NB) * L
        pltpu.async_copy(text_hbm.at[pl.ds(base, IDXG)], ibufs[b], isems[b])

    def fire_gathers(b):
        pltpu.make_async_copy(text_hbm.at[pl.ds(0, IDXG)],
                              ibufs[b], isems[b]).wait()
        ib = ibufs[b]
        for s in range(IDXG // 16):
            ib[pl.ds(s * 16, 16)] = remap(ib[pl.ds(s * 16, 16)])
        for c in range(NCHUNK):
            pltpu.async_copy(tbl_hbm.at[ib.at[pl.ds(c * 128, 128)]],
                             rbufs[b].at[pl.ds(c * 128, 128)], gsems[b])

    def drain_reduce(g, b):
        for c in range(NCHUNK):
            pltpu.make_async_copy(
                tbl_hbm.at[ibufs[b].at[pl.ds(c * 128, 128)]],
                rbufs[b].at[pl.ds(c * 128, 128)], gsems[b]).wait()
        rbuf = rbufs[b]
        for i in range(NB):
            @pl.loop(0, L // 4, init_carry=(zero, zero, zero, zero),
                     unroll=UNROLL)
            def _acc(j, carry):
                a0, a1, a2, a3 = carry
                base = i * L + j * 4
                return (a0 + rbuf[base], a1 + rbuf[base + 1],
                        a2 + rbuf[base + 2], a3 + rbuf[base + 3])

            a0, a1, a2, a3 = _acc
            out_v[g * NB + i, :] = ((a0 + a1) + (a2 + a3)) * inv_l + bvec

    start_idx(0, 0)
    start_idx(1, 1)
    fire_gathers(0)

    @pl.loop(0, GROUPS, step=2)
    def _pair(g):
        fire_gathers(1)
        drain_reduce(g, 0)

        @pl.when(g + 2 < GROUPS)
        def _():
            start_idx(g + 2, 0)

        @pl.when(g + 2 < GROUPS)
        def _():
            fire_gathers(0)

        drain_reduce(g + 1, 1)

        @pl.when(g + 3 < GROUPS)
        def _():
            start_idx(g + 3, 1)

    pltpu.sync_copy(out_v, out_hbm.at[pl.ds(wid * BPW, BPW)])


def kernel(text, text_lengths, emb, W1, b1, W2, b2):
    del text_lengths
    text32 = text.astype(jnp.int32).reshape(B * L)
    embT = emb.T
    NCLS = b2.shape[0]
    W2p = jnp.pad(W2, ((0, 0), (0, NCP - NCLS)))
    b2p = jnp.pad(b2, (0, NCP - NCLS))

    tablec, biasc = pl.pallas_call(
        _table_kernel,
        grid=(NBLK,),
        in_specs=[
            pl.BlockSpec((HID, BMT), lambda i: (0, i)),
            pl.BlockSpec((HID, HID), lambda i: (0, 0)),
            pl.BlockSpec((1, HID), lambda i: (0, 0)),
            pl.BlockSpec((HID, NCP), lambda i: (0, 0)),
            pl.BlockSpec((1, NCP), lambda i: (0, 0)),
        ],
        out_specs=[
            pl.BlockSpec((QROWS, 128), lambda i: (i, 0)),
            pl.BlockSpec((1, NCP), lambda i: (0, 0)),
        ],
        out_shape=[
            jax.ShapeDtypeStruct((VCAP // 8, 128), jnp.float32),
            jax.ShapeDtypeStruct((1, NCP), jnp.float32),
        ],
        compiler_params=pltpu.CompilerParams(fuse_transposed_lhs_in_matmul=True),
    )(embT, W1, b1.reshape(1, HID), W2p, b2p.reshape(1, NCP))
    table2 = tablec.reshape(VCAP, NCP)

    mesh = plsc.VectorSubcoreMesh(core_axis_name="c", subcore_axis_name="s")
    pooled = pl.kernel(
        _pool_kernel,
        out_type=jax.ShapeDtypeStruct((B, NCP), jnp.float32),
        mesh=mesh,
        compiler_params=pltpu.CompilerParams(use_tc_tiling_on_sc=False),
        scratch_types=[
            pltpu.VMEM((IDXG,), jnp.int32),
            pltpu.VMEM((IDXG,), jnp.int32),
            pltpu.VMEM((IDXG, NCP), jnp.float32),
            pltpu.VMEM((IDXG, NCP), jnp.float32),
            pltpu.VMEM((BPW, NCP), jnp.float32),
            pltpu.VMEM((1, NCP), jnp.float32),
            pltpu.SemaphoreType.DMA,
            pltpu.SemaphoreType.DMA,
            pltpu.SemaphoreType.DMA,
            pltpu.SemaphoreType.DMA,
        ],
    )(text32, table2, biasc)

    return pooled[:, :NCLS]


# speedup vs baseline: 1.1881x; 1.0067x over previous
import functools

import jax
import jax.numpy as jnp
from jax import lax
from jax.experimental import pallas as pl
from jax.experimental.pallas import tpu as pltpu
from jax.experimental.pallas import tpu_sc as plsc

B = 16384
L = 200
VOCAB = 1000000
HID = 32
NCP = 16
BMT = 32768
NBLK = 31
VCAP = NBLK * BMT
QROWS = BMT // 8
NC = 2
NS = 16
NW = NC * NS
BPW = B // NW
NB = 16
GROUPS = BPW // NB
IDXG = NB * L
NCHUNK = IDXG // 128
UNROLL = 10


def _table_kernel(embT_ref, w1_ref, b1_ref, w2p_ref, b2p_ref, o_ref, bias_ref):
    wc = jnp.dot(w1_ref[...], w2p_ref[...],
                 preferred_element_type=jnp.float32)
    bias_ref[...] = jnp.dot(b1_ref[...], w2p_ref[...],
                            preferred_element_type=jnp.float32) + b2p_ref[...]
    wcb = wc.astype(jnp.bfloat16)
    for q in range(8):
        eb = embT_ref[:, q * QROWS:(q + 1) * QROWS].astype(jnp.bfloat16)
        res_q = jnp.dot(eb.T, wcb,
                        preferred_element_type=jnp.float32)
        o_ref[:, q * NCP:(q + 1) * NCP] = res_q


def _pool_kernel(text_hbm, tbl_hbm, bias_hbm, out_hbm,
                 ibuf0, ibuf1, rbuf0, rbuf1, out_v, bias_v,
                 isem0, isem1, gsem0, gsem1):
    wid = lax.axis_index("s") * NC + lax.axis_index("c")
    inv_l = jnp.float32(1.0 / L)
    zero = jnp.zeros((16,), jnp.float32)
    pltpu.sync_copy(bias_hbm, bias_v)
    bvec = bias_v[0, :]
    ibufs = (ibuf0, ibuf1)
    rbufs = (rbuf0, rbuf1)
    isems = (isem0, isem1)
    gsems = (gsem0, gsem1)

    def remap(v):
        return (v & -BMT) | ((v & (QROWS - 1)) << 3) | ((v >> 12) & 7)

    def start_idx(g, b):
        base = (wid * BPW + g * NB) * L
        pltpu.async_copy(text_hbm.at[pl.ds(base, IDXG)], ibufs[b], isems[b])

    def fire_gathers(b):
        pltpu.make_async_copy(text_hbm.at[pl.ds(0, IDXG)],
                              ibufs[b], isems[b]).wait()
        ib = ibufs[b]
        for s in range(IDXG // 16):
            ib[pl.ds(s * 16, 16)] = remap(ib[pl.ds(s * 16, 16)])
        for c in range(NCHUNK):
            pltpu.async_copy(tbl_hbm.at[ib.at[pl.ds(c * 128, 128)]],
                             rbufs[b].at[pl.ds(c * 128, 128)], gsems[b])

    def drain_reduce(g, b):
        for c in range(NCHUNK):
            pltpu.make_async_copy(
                tbl_hbm.at[ibufs[b].at[pl.ds(c * 128, 128)]],
                rbufs[b].at[pl.ds(c * 128, 128)], gsems[b]).wait()
        rbuf = rbufs[b]
        for i in range(NB):
            @pl.loop(0, L // 4, init_carry=(zero, zero, zero, zero),
                     unroll=UNROLL)
            def _acc(j, carry):
                a0, a1, a2, a3 = carry
                base = i * L + j * 4
                return (a0 + rbuf[base], a1 + rbuf[base + 1],
                        a2 + rbuf[base + 2], a3 + rbuf[base + 3])

            a0, a1, a2, a3 = _acc
            out_v[g * NB + i, :] = ((a0 + a1) + (a2 + a3)) * inv_l + bvec

    start_idx(0, 0)
    start_idx(1, 1)
    fire_gathers(0)

    @pl.loop(0, GROUPS, step=2)
    def _pair(g):
        fire_gathers(1)
        drain_reduce(g, 0)

        @pl.when(g + 2 < GROUPS)
        def _():
            start_idx(g + 2, 0)

        @pl.when(g + 2 < GROUPS)
        def _():
            fire_gathers(0)

        drain_reduce(g + 1, 1)

        @pl.when(g + 3 < GROUPS)
        def _():
            start_idx(g + 3, 1)

    pltpu.sync_copy(out_v, out_hbm.at[pl.ds(wid * BPW, BPW)])


def kernel(text, text_lengths, emb, W1, b1, W2, b2):
    del text_lengths
    text32 = text.astype(jnp.int32).reshape(B * L)
    embT = emb.T
    NCLS = b2.shape[0]
    W2p = jnp.pad(W2, ((0, 0), (0, NCP - NCLS)))
    b2p = jnp.pad(b2, (0, NCP - NCLS))

    tablec, biasc = pl.pallas_call(
        _table_kernel,
        grid=(NBLK,),
        in_specs=[
            pl.BlockSpec((HID, BMT), lambda i: (0, i)),
            pl.BlockSpec((HID, HID), lambda i: (0, 0)),
            pl.BlockSpec((1, HID), lambda i: (0, 0)),
            pl.BlockSpec((HID, NCP), lambda i: (0, 0)),
            pl.BlockSpec((1, NCP), lambda i: (0, 0)),
        ],
        out_specs=[
            pl.BlockSpec((QROWS, 128), lambda i: (i, 0)),
            pl.BlockSpec((1, NCP), lambda i: (0, 0)),
        ],
        out_shape=[
            jax.ShapeDtypeStruct((VCAP // 8, 128), jnp.float32),
            jax.ShapeDtypeStruct((1, NCP), jnp.float32),
        ],
        compiler_params=pltpu.CompilerParams(fuse_transposed_lhs_in_matmul=True),
    )(embT, W1, b1.reshape(1, HID), W2p, b2p.reshape(1, NCP))
    table2 = tablec.reshape(VCAP, NCP)

    mesh = plsc.VectorSubcoreMesh(core_axis_name="c", subcore_axis_name="s")
    pooled = pl.kernel(
        _pool_kernel,
        out_type=jax.ShapeDtypeStruct((B, NCP), jnp.float32),
        mesh=mesh,
        compiler_params=pltpu.CompilerParams(use_tc_tiling_on_sc=False),
        scratch_types=[
            pltpu.VMEM((IDXG,), jnp.int32),
            pltpu.VMEM((IDXG,), jnp.int32),
            pltpu.VMEM((IDXG, NCP), jnp.float32),
            pltpu.VMEM((IDXG, NCP), jnp.float32),
            pltpu.VMEM((BPW, NCP), jnp.float32),
            pltpu.VMEM((1, NCP), jnp.float32),
            pltpu.SemaphoreType.DMA,
            pltpu.SemaphoreType.DMA,
            pltpu.SemaphoreType.DMA,
            pltpu.SemaphoreType.DMA,
        ],
    )(text32, table2, biasc)

    return pooled[:, :NCLS]
